# Initial kernel scaffold; baseline (speedup 1.0000x reference)
import jax
import jax.numpy as jnp
from jax.experimental import pallas as pl


def kernel(x, batch_index):
    raise NotImplementedError("write your pallas kernel here")


# speedup vs baseline: 3.4215x; 3.4215x over previous
import functools

import jax
import jax.numpy as jnp
from jax import lax
from jax.experimental import pallas as pl
from jax.experimental.pallas import tpu as pltpu
from jax.experimental.pallas import tpu_sc as plsc

N_ROWS = 100000
D = 128
NSEG = 64
G = 128
NC = 2
NS = 16
NW = NC * NS
ROWS_PER_W = N_ROWS // NW


def _tc_finish(sums_ref, cnts_ref, out_ref):
    s = sums_ref[0] + sums_ref[1]
    c = cnts_ref[0] + cnts_ref[1]
    out_ref[...] = s / jnp.maximum(c, 1.0)


def kernel(x, batch_index):
    bi = batch_index.astype(jnp.int32)
    mesh = plsc.VectorSubcoreMesh(core_axis_name="c", subcore_axis_name="s")

    @functools.partial(
        pl.kernel,
        mesh=mesh,
        out_type=[
            jax.ShapeDtypeStruct((NC, NSEG, D), jnp.float32),
            jax.ShapeDtypeStruct((NC, NSEG, D), jnp.float32),
        ],
        scratch_types=[
            pltpu.VMEM((G, D), jnp.float32),
            pltpu.VMEM((G,), jnp.int32),
            pltpu.VMEM((G, D), jnp.float32),
            pltpu.VMEM((8, D), jnp.float32),
            pltpu.VMEM((8,), jnp.int32),
            pltpu.VMEM((4, D), jnp.float32),
            pltpu.VMEM_SHARED((NSEG, D), jnp.float32),
            pltpu.VMEM_SHARED((NSEG, D), jnp.float32),
        ],
    )
    def sc_seg(x_hbm, bi_hbm, sums_out, cnts_out,
               rows_v, idx_v, ones_v, rows8_v, idx8_v, z_v,
               sums_sh, cnts_sh):
        c = lax.axis_index("c")
        s = lax.axis_index("s")
        wid = c * NS + s

        zero16 = jnp.zeros((16,), jnp.float32)
        one16 = jnp.ones((16,), jnp.float32)
        for r in range(4):
            for j in range(D // 16):
                z_v[r, pl.ds(j * 16, 16)] = zero16

        def ones_body(r, _):
            for j in range(D // 16):
                ones_v[r, pl.ds(j * 16, 16)] = one16
            return _

        lax.fori_loop(0, G, ones_body, None)

        pltpu.sync_copy(z_v, sums_sh.at[pl.ds(s * 4, 4)])
        pltpu.sync_copy(z_v, cnts_sh.at[pl.ds(s * 4, 4)])
        plsc.subcore_barrier()

        start = (wid * ROWS_PER_W) & -8
        end = jnp.where(wid == NW - 1, N_ROWS, ((wid + 1) * ROWS_PER_W) & -8)
        n_big = (end - start) // G
        tail0 = start + n_big * G
        n_tail = (end - tail0) // 8

        def big_body(g, _):
            off = pl.multiple_of(start + g * G, 8)
            pltpu.sync_copy(x_hbm.at[pl.ds(off, G)], rows_v)
            pltpu.sync_copy(bi_hbm.at[pl.ds(off, G)], idx_v)
            pltpu.sync_copy(rows_v, sums_sh.at[idx_v], add=True)
            pltpu.sync_copy(ones_v, cnts_sh.at[idx_v], add=True)
            return _

        lax.fori_loop(0, n_big, big_body, None)

        def tail_body(t, _):
            off = pl.multiple_of(tail0 + t * 8, 8)
            pltpu.sync_copy(x_hbm.at[pl.ds(off, 8)], rows8_v)
            pltpu.sync_copy(bi_hbm.at[pl.ds(off, 8)], idx8_v)
            pltpu.sync_copy(rows8_v, sums_sh.at[idx8_v], add=True)
            pltpu.sync_copy(ones_v.at[pl.ds(0, 8)], cnts_sh.at[idx8_v], add=True)
            return _

        lax.fori_loop(0, n_tail, tail_body, None)

        plsc.subcore_barrier()

        @pl.when(s == 0)
        def _():
            pltpu.sync_copy(sums_sh, sums_out.at[c])
            pltpu.sync_copy(cnts_sh, cnts_out.at[c])

    sums, cnts = sc_seg(x, bi)
    out = pl.pallas_call(
        _tc_finish,
        out_shape=jax.ShapeDtypeStruct((NSEG, D), jnp.float32),
    )(sums, cnts)
    return out


# speedup vs baseline: 4.2205x; 1.2335x over previous
import functools

import jax
import jax.numpy as jnp
from jax import lax
from jax.experimental import pallas as pl
from jax.experimental.pallas import tpu as pltpu
from jax.experimental.pallas import tpu_sc as plsc

N_ROWS = 100000
D = 128
NSEG = 64
G = 128
NC = 2
NS = 16
NW = NC * NS
ROWS_PER_W = N_ROWS // NW
N_BIG = (ROWS_PER_W - 8) // G
NBUF = 3


def _tc_finish(sums_ref, cnts_ref, out_ref):
    s = sums_ref[0] + sums_ref[1]
    c = cnts_ref[0] + cnts_ref[1]
    out_ref[...] = s / jnp.maximum(c, 1.0)


def kernel(x, batch_index):
    bi = batch_index.astype(jnp.int32)
    mesh = plsc.VectorSubcoreMesh(core_axis_name="c", subcore_axis_name="s")

    @functools.partial(
        pl.kernel,
        mesh=mesh,
        out_type=[
            jax.ShapeDtypeStruct((NC, NSEG, D), jnp.float32),
            jax.ShapeDtypeStruct((NC, NSEG, D), jnp.float32),
        ],
        scratch_types=(
            [pltpu.VMEM((G, D), jnp.float32) for _ in range(NBUF)]
            + [pltpu.VMEM((G,), jnp.int32) for _ in range(NBUF)]
            + [
                pltpu.VMEM((G, D), jnp.float32),
                pltpu.VMEM((8, D), jnp.float32),
                pltpu.VMEM((8,), jnp.int32),
                pltpu.VMEM((4, D), jnp.float32),
                pltpu.VMEM_SHARED((NSEG, D), jnp.float32),
                pltpu.VMEM_SHARED((NSEG, D), jnp.float32),
            ]
            + [pltpu.SemaphoreType.DMA for _ in range(4 * NBUF)]
        ),
    )
    def sc_seg(x_hbm, bi_hbm, sums_out, cnts_out,
               rows0, rows1, rows2, idx0, idx1, idx2,
               ones_v, rows8_v, idx8_v, z_v, sums_sh, cnts_sh,
               gr0, gr1, gr2, gi0, gi1, gi2,
               ss0, ss1, ss2, sc0, sc1, sc2):
        rows_b = (rows0, rows1, rows2)
        idx_b = (idx0, idx1, idx2)
        sem_gr = (gr0, gr1, gr2)
        sem_gi = (gi0, gi1, gi2)
        sem_s = (ss0, ss1, ss2)
        sem_c = (sc0, sc1, sc2)

        c = lax.axis_index("c")
        s = lax.axis_index("s")
        wid = c * NS + s

        zero16 = jnp.zeros((16,), jnp.float32)
        one16 = jnp.ones((16,), jnp.float32)
        for r in range(4):
            for j in range(D // 16):
                z_v[r, pl.ds(j * 16, 16)] = zero16

        def ones_body(r, carry):
            for j in range(D // 16):
                ones_v[r, pl.ds(j * 16, 16)] = one16
            return carry

        lax.fori_loop(0, G, ones_body, None)

        pltpu.sync_copy(z_v, sums_sh.at[pl.ds(s * 4, 4)])
        pltpu.sync_copy(z_v, cnts_sh.at[pl.ds(s * 4, 4)])
        plsc.subcore_barrier()

        start = (wid * ROWS_PER_W) & -8
        end = jnp.where(wid == NW - 1, N_ROWS, ((wid + 1) * ROWS_PER_W) & -8)
        tail0 = start + N_BIG * G
        n_tail = (end - tail0) // 8

        gathers = {}
        scatters = {}

        def issue_gather(g):
            b = g % NBUF
            off = pl.multiple_of(start + g * G, 8)
            gathers[g] = (
                pltpu.async_copy(x_hbm.at[pl.ds(off, G)], rows_b[b], sem_gr[b]),
                pltpu.async_copy(bi_hbm.at[pl.ds(off, G)], idx_b[b], sem_gi[b]),
            )

        issue_gather(0)
        issue_gather(1)
        for g in range(N_BIG):
            b = g % NBUF
            for d in gathers.pop(g):
                d.wait()
            scatters[g] = (
                pltpu.async_copy(
                    rows_b[b], sums_sh.at[idx_b[b]], sem_s[b], add=True),
                pltpu.async_copy(
                    ones_v, cnts_sh.at[idx_b[b]], sem_c[b], add=True),
            )
            if g + 2 < N_BIG:
                if g >= 1:
                    for d in scatters.pop(g - 1):
                        d.wait()
                issue_gather(g + 2)
        for g in sorted(scatters):
            for d in scatters.pop(g):
                d.wait()

        def tail_body(t, carry):
            off = pl.multiple_of(tail0 + t * 8, 8)
            pltpu.sync_copy(x_hbm.at[pl.ds(off, 8)], rows8_v)
            pltpu.sync_copy(bi_hbm.at[pl.ds(off, 8)], idx8_v)
            pltpu.sync_copy(rows8_v, sums_sh.at[idx8_v], add=True)
            pltpu.sync_copy(
                ones_v.at[pl.ds(0, 8)], cnts_sh.at[idx8_v], add=True)
            return carry

        lax.fori_loop(0, n_tail, tail_body, None)

        plsc.subcore_barrier()

        @pl.when(s == 0)
        def _():
            pltpu.sync_copy(sums_sh, sums_out.at[c])
            pltpu.sync_copy(cnts_sh, cnts_out.at[c])

    sums, cnts = sc_seg(x, bi)
    out = pl.pallas_call(
        _tc_finish,
        out_shape=jax.ShapeDtypeStruct((NSEG, D), jnp.float32),
    )(sums, cnts)
    return out


# speedup vs baseline: 5.6052x; 1.3281x over previous
import functools

import jax
import jax.numpy as jnp
from jax import lax
from jax.experimental import pallas as pl
from jax.experimental.pallas import tpu as pltpu
from jax.experimental.pallas import tpu_sc as plsc

N_ROWS = 100000
D = 128
NSEG = 64
G = 128
NC = 2
NS = 16
NW = NC * NS
ROWS_PER_W = N_ROWS // NW
N_BIG = (ROWS_PER_W - 8) // G
NBUF = 3
BC_COLS = 12544
BC_GRID = BC_COLS // 128


def _tc_bincount(bi_ref, cnt_ref, acc_ref):
    i = pl.program_id(0)

    @pl.when(i == 0)
    def _():
        acc_ref[...] = jnp.zeros((NSEG, D), jnp.float32)

    blk = bi_ref[...]
    seg = lax.broadcasted_iota(jnp.int32, (NSEG, D), 0)
    tot = jnp.zeros((NSEG, D), jnp.float32)
    for r in range(8):
        row = jnp.broadcast_to(blk[r:r + 1, :], (NSEG, D))
        tot = tot + (row == seg).astype(jnp.float32)
    acc_ref[...] += tot

    @pl.when(i == BC_GRID - 1)
    def _():
        cnt_ref[...] = acc_ref[...]


def _tc_finish(sums_ref, cnts_ref, out_ref):
    s = sums_ref[0] + sums_ref[1]
    c = jnp.sum(cnts_ref[...], axis=1, keepdims=True)
    out_ref[...] = s / jnp.maximum(c, 1.0)


def kernel(x, batch_index):
    bi = batch_index.astype(jnp.int32)
    mesh = plsc.VectorSubcoreMesh(core_axis_name="c", subcore_axis_name="s")

    @functools.partial(
        pl.kernel,
        mesh=mesh,
        out_type=jax.ShapeDtypeStruct((NC, NSEG, D), jnp.float32),
        scratch_types=(
            [pltpu.VMEM((G, D), jnp.float32) for _ in range(NBUF)]
            + [pltpu.VMEM((G,), jnp.int32) for _ in range(NBUF)]
            + [
                pltpu.VMEM((8, D), jnp.float32),
                pltpu.VMEM((8,), jnp.int32),
                pltpu.VMEM((4, D), jnp.float32),
                pltpu.VMEM_SHARED((NSEG, D), jnp.float32),
            ]
            + [pltpu.SemaphoreType.DMA for _ in range(3 * NBUF)]
        ),
    )
    def sc_seg(x_hbm, bi_hbm, sums_out,
               rows0, rows1, rows2, idx0, idx1, idx2,
               rows8_v, idx8_v, z_v, sums_sh,
               gr0, gr1, gr2, gi0, gi1, gi2, ss0, ss1, ss2):
        rows_b = (rows0, rows1, rows2)
        idx_b = (idx0, idx1, idx2)
        sem_gr = (gr0, gr1, gr2)
        sem_gi = (gi0, gi1, gi2)
        sem_s = (ss0, ss1, ss2)

        c = lax.axis_index("c")
        s = lax.axis_index("s")
        wid = c * NS + s

        zero16 = jnp.zeros((16,), jnp.float32)
        for r in range(4):
            for j in range(D // 16):
                z_v[r, pl.ds(j * 16, 16)] = zero16

        pltpu.sync_copy(z_v, sums_sh.at[pl.ds(s * 4, 4)])
        plsc.subcore_barrier()

        start = (wid * ROWS_PER_W) & -8
        end = jnp.where(wid == NW - 1, N_ROWS, ((wid + 1) * ROWS_PER_W) & -8)
        tail0 = start + N_BIG * G
        n_tail = (end - tail0) // 8

        gathers = {}
        scatters = {}

        def issue_gather(g):
            b = g % NBUF
            off = pl.multiple_of(start + g * G, 8)
            gathers[g] = (
                pltpu.async_copy(x_hbm.at[pl.ds(off, G)], rows_b[b], sem_gr[b]),
                pltpu.async_copy(bi_hbm.at[pl.ds(off, G)], idx_b[b], sem_gi[b]),
            )

        issue_gather(0)
        issue_gather(1)
        for g in range(N_BIG):
            b = g % NBUF
            for d in gathers.pop(g):
                d.wait()
            scatters[g] = pltpu.async_copy(
                rows_b[b], sums_sh.at[idx_b[b]], sem_s[b], add=True)
            if g + 2 < N_BIG:
                if g >= 1:
                    scatters.pop(g - 1).wait()
                issue_gather(g + 2)
        for g in sorted(scatters):
            scatters.pop(g).wait()

        def tail_body(t, carry):
            off = pl.multiple_of(tail0 + t * 8, 8)
            pltpu.sync_copy(x_hbm.at[pl.ds(off, 8)], rows8_v)
            pltpu.sync_copy(bi_hbm.at[pl.ds(off, 8)], idx8_v)
            pltpu.sync_copy(rows8_v, sums_sh.at[idx8_v], add=True)
            return carry

        lax.fori_loop(0, n_tail, tail_body, None)

        plsc.subcore_barrier()

        @pl.when(s == 0)
        def _():
            pltpu.sync_copy(sums_sh, sums_out.at[c])

    sums = sc_seg(x, bi)

    bi2d = jnp.pad(bi, (0, 8 * BC_COLS - N_ROWS),
                   constant_values=NSEG).reshape(8, BC_COLS)
    cnts = pl.pallas_call(
        _tc_bincount,
        grid=(BC_GRID,),
        in_specs=[pl.BlockSpec((8, 128), lambda i: (0, i))],
        out_specs=pl.BlockSpec((NSEG, D), lambda i: (0, 0)),
        out_shape=jax.ShapeDtypeStruct((NSEG, D), jnp.float32),
        scratch_shapes=[pltpu.VMEM((NSEG, D), jnp.float32)],
    )(bi2d)

    out = pl.pallas_call(
        _tc_finish,
        out_shape=jax.ShapeDtypeStruct((NSEG, D), jnp.float32),
    )(sums, cnts)
    return out
